# Initial kernel scaffold; baseline (speedup 1.0000x reference)
import jax
import jax.numpy as jnp
from jax.experimental import pallas as pl


def kernel(x, edge_index, edge_attr, W_in, b_in, W_msg, W_edge, W_self, nm_W1, nm_b1, nm_W2, nm_b2, node_mask_token, em_W1, em_b1, em_W2, em_b2, edge_mask_token):
    raise NotImplementedError("write your pallas kernel here")


# speedup vs baseline: 73.8042x; 73.8042x over previous
import functools

import jax
import jax.numpy as jnp
import numpy as np
from jax import lax
from jax.experimental import pallas as pl
from jax.experimental.pallas import tpu as pltpu
from jax.experimental.pallas import tpu_sc as plsc

N = 10000
NP = 10112
E = 320000
D = 128
NMASK = 1500
EMASK = 48000
NM_PAD = 1536
EM_PAD = 48128

EPT = E // 16


def _tf_block(k0, k1, x1):
    x0 = np.zeros_like(x1)
    x1 = x1.copy()
    ks = [np.uint32(k0), np.uint32(k1),
          np.uint32(k0) ^ np.uint32(k1) ^ np.uint32(0x1BD11BDA)]
    rots = [(13, 15, 26, 6), (17, 29, 16, 24)]
    x0 = x0 + ks[0]
    x1 = x1 + ks[1]
    for i in range(5):
        for d in rots[i % 2]:
            x0 = x0 + x1
            x1 = ((x1 << np.uint32(d)) | (x1 >> np.uint32(32 - d))) ^ x0
        x0 = x0 + ks[(i + 1) % 3]
        x1 = x1 + ks[(i + 2) % 3] + np.uint32(i + 1)
    return x0, x1


def _np_permutation(seed, n):
    key = (np.uint32(0), np.uint32(seed))
    num_rounds = int(np.ceil(3 * np.log(max(1, n)) / np.log(0xFFFFFFFF)))
    x = np.arange(n, dtype=np.int32)
    for _ in range(num_rounds):
        b1, b2 = _tf_block(key[0], key[1], np.arange(2, dtype=np.uint32))
        key, sub = (b1[0], b2[0]), (b1[1], b2[1])
        c1, c2 = _tf_block(sub[0], sub[1], np.arange(n, dtype=np.uint32))
        x = x[np.argsort(c1 ^ c2, kind='stable')]
    return x


@functools.cache
def _mask_indices():
    idx_n = _np_permutation(123, N)[:NMASK]
    idx_e = _np_permutation(456, E)[:EMASK]
    idx_n_pad = np.concatenate(
        [idx_n, np.full(NM_PAD - NMASK, idx_n[0], np.int32)])
    idx_e_pad = np.concatenate(
        [idx_e, np.full(EM_PAD - EMASK, idx_e[0], np.int32)])
    eflag = np.zeros((E, 1), np.bool_)
    eflag[idx_e] = True
    nflag = np.zeros((N, 1), np.bool_)
    nflag[idx_n] = True
    rowids = (idx_e_pad // 32).astype(np.int32)
    off8 = np.repeat(((idx_e_pad % 32) * 4)[:, None], 8,
                     axis=1).astype(np.int32)
    return idx_n_pad, idx_e_pad, eflag, nflag, rowids, off8


@functools.cache
def _mesh():
    return plsc.VectorSubcoreMesh(core_axis_name="c", subcore_axis_name="s")


def _wid():
    return lax.axis_index("s") * 2 + lax.axis_index("c")


def _zeros16():
    return jnp.broadcast_to(jnp.float32(0.0), (16,))


def _k1_body(x_h, ep_h, idn_h, idr_h, xg_h, g_h,
             idx48, row48, idx128, grow128, idx96, grow96, sem):
    w = _wid()
    nb = w * 48
    pltpu.sync_copy(idn_h.at[pl.ds(nb, 48)], idx48)
    pltpu.async_copy(x_h.at[idx48], row48, sem).wait()
    pltpu.sync_copy(row48, xg_h.at[pl.ds(nb, 48)])

    eb = w * 1504

    @pl.loop(0, 11)
    def _(i):
        b = eb + i * 128
        pltpu.sync_copy(idr_h.at[pl.ds(b, 128)], idx128)
        pltpu.async_copy(ep_h.at[idx128], grow128, sem).wait()
        pltpu.sync_copy(grow128, g_h.at[pl.ds(b, 128)])

    b = eb + 11 * 128
    pltpu.sync_copy(idr_h.at[pl.ds(b, 96)], idx96)
    pltpu.async_copy(ep_h.at[idx96], grow96, sem).wait()
    pltpu.sync_copy(grow96, g_h.at[pl.ds(b, 96)])


@functools.cache
def _k1():
    return pl.kernel(
        _k1_body,
        mesh=_mesh(),
        out_type=(jax.ShapeDtypeStruct((NM_PAD, D), jnp.float32),
                  jax.ShapeDtypeStruct((EM_PAD, D), jnp.int32)),
        scratch_types=[pltpu.VMEM((48,), jnp.int32),
                       pltpu.VMEM((48, D), jnp.float32),
                       pltpu.VMEM((128,), jnp.int32),
                       pltpu.VMEM((128, D), jnp.int32),
                       pltpu.VMEM((96,), jnp.int32),
                       pltpu.VMEM((96, D), jnp.int32),
                       pltpu.SemaphoreType.DMA])


def _kx_body(g_ref, off_ref, o_ref):
    g = g_ref[...]
    off = off_ref[:, 0:1]
    lane = lax.broadcasted_iota(jnp.int32, g.shape, 1)
    out = jnp.zeros_like(g)
    for j in range(4):
        fj = jnp.sum(jnp.where(lane == off + j, g, 0), axis=1, keepdims=True)
        out = out + jnp.where(lane == j, fj, 0)
    o_ref[...] = out


def _k2a_body(x_ref, w_ref, b_ref, o_ref):
    o_ref[...] = jnp.maximum(
        jnp.dot(x_ref[...], w_ref[...], preferred_element_type=jnp.float32)
        + b_ref[...], 0.0)


def _k4_body(h_h, hm_h, idn_h, src_h, dst_h,
             A_h, Amg_h,
             accA, zb, sidx, didx, rows,
             sidx_t, didx_t, rows_t, hrbuf, hidx, sem):
    c = lax.axis_index("c")
    s = lax.axis_index("s")
    z16 = _zeros16()
    r0 = s * 632
    nb = s * 96

    @pl.loop(0, 128)
    def _(r):
        for k in range(8):
            zb[r, pl.ds(k * 16, 16)] = z16

    for j in range(4):
        pltpu.sync_copy(zb, accA.at[pl.ds(r0 + j * 128, 128)])
    pltpu.sync_copy(zb.at[pl.ds(0, 120)], accA.at[pl.ds(r0 + 512, 120)])

    pltpu.sync_copy(idn_h.at[pl.ds(nb, 96)], hidx)

    plsc.subcore_barrier()

    def edge_pass(tbl):
        e0 = s * EPT

        def chunk(b, si, di, rw):
            pltpu.sync_copy(src_h.at[pl.ds(b, si.shape[0])], si)
            pltpu.sync_copy(dst_h.at[pl.ds(b, di.shape[0])], di)
            pltpu.async_copy(tbl.at[si], rw, sem).wait()
            pltpu.sync_copy(rw, accA.at[di], add=True)

        @pl.loop(0, 156)
        def _(i):
            chunk(e0 + i * 128, sidx, didx, rows)

        chunk(e0 + 156 * 128, sidx_t, didx_t, rows_t)

    @pl.when(c == 0)
    def _():
        edge_pass(h_h)

    @pl.when(c == 1)
    def _():
        edge_pass(hm_h)

    plsc.subcore_barrier()

    @pl.when(c == 0)
    def _():
        pltpu.sync_copy(accA.at[pl.ds(r0, 632)], A_h.at[pl.ds(r0, 632)])

    @pl.when(c == 1)
    def _():
        pltpu.sync_copy(accA.at[hidx], hrbuf)
        pltpu.sync_copy(hrbuf, Amg_h.at[pl.ds(nb, 96)])


@functools.cache
def _k4():
    return pl.kernel(
        _k4_body,
        mesh=_mesh(),
        out_type=(jax.ShapeDtypeStruct((NP, D), jnp.float32),
                  jax.ShapeDtypeStruct((NM_PAD, D), jnp.float32)),
        scratch_types=[pltpu.VMEM_SHARED((NP, D), jnp.float32),
                       pltpu.VMEM((128, D), jnp.float32),
                       pltpu.VMEM((128,), jnp.int32),
                       pltpu.VMEM((128,), jnp.int32),
                       pltpu.VMEM((128, D), jnp.float32),
                       pltpu.VMEM((32,), jnp.int32),
                       pltpu.VMEM((32,), jnp.int32),
                       pltpu.VMEM((32, D), jnp.float32),
                       pltpu.VMEM((96, D), jnp.float32),
                       pltpu.VMEM((96,), jnp.int32),
                       pltpu.SemaphoreType.DMA])


def _k4b_body(dst_h, ea_h, eam_h,
              Bm_h, B_h,
              accB, zb, bufw, didx, eav):
    c = lax.axis_index("c")
    s = lax.axis_index("s")
    z16 = _zeros16()
    r0 = s * 632
    nb = s * 96

    @pl.loop(0, 128)
    def _(r):
        for k in range(8):
            zb[r, pl.ds(k * 16, 16)] = z16
            bufw[r, pl.ds(k * 16, 16)] = z16

    for j in range(4):
        pltpu.sync_copy(zb, accB.at[pl.ds(r0 + j * 128, 128)])
    pltpu.sync_copy(zb.at[pl.ds(0, 120)], accB.at[pl.ds(r0 + 512, 120)])
    plsc.subcore_barrier()

    def sweep(ea):
        def chunk(j):
            b = j * 128
            pltpu.sync_copy(dst_h.at[pl.ds(b, 128)], didx)
            pltpu.sync_copy(ea.at[pl.ds(j * 16, 16)], eav)

            @pl.loop(0, 128)
            def _(r):
                bufw[r, pl.ds(0, 16)] = eav[r // 8, pl.ds((r % 8) * 16, 16)]

            pltpu.sync_copy(bufw, accB.at[didx], add=True)

        @pl.loop(0, 156)
        def _(i):
            chunk(i * 16 + s)

        @pl.when(s < 4)
        def _():
            chunk(2496 + s)

    @pl.when(c == 0)
    def _():
        sweep(ea_h)

    @pl.when(c == 1)
    def _():
        sweep(eam_h)

    plsc.subcore_barrier()

    @pl.when(c == 0)
    def _():
        pltpu.sync_copy(accB.at[pl.ds(r0, 632)], B_h.at[pl.ds(r0, 632)])

    @pl.when(c == 1)
    def _():
        pltpu.sync_copy(accB.at[pl.ds(r0, 632)], Bm_h.at[pl.ds(r0, 632)])


@functools.cache
def _k4b():
    return pl.kernel(
        _k4b_body,
        mesh=_mesh(),
        out_type=(jax.ShapeDtypeStruct((NP, D), jnp.float32),
                  jax.ShapeDtypeStruct((NP, D), jnp.float32)),
        scratch_types=[pltpu.VMEM_SHARED((NP, D), jnp.float32),
                       pltpu.VMEM((128, D), jnp.float32),
                       pltpu.VMEM((128, D), jnp.float32),
                       pltpu.VMEM((128,), jnp.int32),
                       pltpu.VMEM((16, D), jnp.float32)])


def _k5_body(a_ref, bm_ref, h_ref, wm_ref, we_ref, ws_ref, w1a_ref, w1b_ref,
             p_ref, q_ref):
    emb = jnp.maximum(
        jnp.dot(a_ref[...], wm_ref[...], preferred_element_type=jnp.float32)
        + jnp.dot(bm_ref[...], we_ref[...], preferred_element_type=jnp.float32)
        + jnp.dot(h_ref[...], ws_ref[...], preferred_element_type=jnp.float32),
        0.0)
    p_ref[...] = jnp.dot(emb, w1a_ref[...], preferred_element_type=jnp.float32)
    q_ref[...] = jnp.dot(emb, w1b_ref[...], preferred_element_type=jnp.float32)


def _k6b_body(sm_h, dm_h, p_h, q_h, hm_h, b_h, idn_h, r_h, hmg_h, bg_h,
              pv, si, di, rp, rq, pv96, si96, di96, rp96, rq96,
              idx48, buf48, sem):
    w = _wid()
    nb = w * 48
    pltpu.sync_copy(idn_h.at[pl.ds(nb, 48)], idx48)
    pltpu.async_copy(hm_h.at[idx48], buf48, sem).wait()
    pltpu.sync_copy(buf48, hmg_h.at[pl.ds(nb, 48)])
    pltpu.async_copy(b_h.at[idx48], buf48, sem).wait()
    pltpu.sync_copy(buf48, bg_h.at[pl.ds(nb, 48)])
    eb = w * 1504

    def chunk(b, pvb, sib, dib, bp, bq, nrow, ngrp):
        pltpu.sync_copy(sm_h.at[pl.ds(b, nrow)], sib)
        pltpu.sync_copy(dm_h.at[pl.ds(b, nrow)], dib)
        pltpu.async_copy(p_h.at[sib], bp, sem).wait()
        pltpu.async_copy(q_h.at[dib], bq, sem).wait()

        @pl.loop(0, nrow)
        def _(r):
            for k in range(8):
                sl = pl.ds(k * 16, 16)
                bp[r, sl] = bp[r, sl] + bq[r, sl]

        pltpu.sync_copy(bp, r_h.at[pl.ds(b, nrow)])

    @pl.loop(0, 11)
    def _(i):
        chunk(eb + i * 128, pv, si, di, rp, rq, 128, 8)

    chunk(eb + 11 * 128, pv96, si96, di96, rp96, rq96, 96, 6)


@functools.cache
def _k6b():
    return pl.kernel(
        _k6b_body,
        mesh=_mesh(),
        out_type=(jax.ShapeDtypeStruct((EM_PAD, D), jnp.float32),
                  jax.ShapeDtypeStruct((NM_PAD, D), jnp.float32),
                  jax.ShapeDtypeStruct((NM_PAD, D), jnp.float32)),
        scratch_types=[pltpu.VMEM((128,), jnp.int32),
                       pltpu.VMEM((128,), jnp.int32),
                       pltpu.VMEM((128,), jnp.int32),
                       pltpu.VMEM((128, D), jnp.float32),
                       pltpu.VMEM((128, D), jnp.float32),
                       pltpu.VMEM((96,), jnp.int32),
                       pltpu.VMEM((96,), jnp.int32),
                       pltpu.VMEM((96,), jnp.int32),
                       pltpu.VMEM((96, D), jnp.float32),
                       pltpu.VMEM((96, D), jnp.float32),
                       pltpu.VMEM((48,), jnp.int32),
                       pltpu.VMEM((48, D), jnp.float32),
                       pltpu.SemaphoreType.DMA])


_EBLK = 512
_NEB = EM_PAD // _EBLK


def _k7_body(amg_ref, bg_ref, hmr_ref, oinj_ref, wm_ref, we_ref, ws_ref,
             nw1_ref, nb1_ref, nw2_ref, nb2_ref,
             r_ref, oef_ref, eb1_ref, ew2_ref, eb2_ref,
             out_ref, acc):
    pid = pl.program_id(0)

    @pl.when(pid == 0)
    def _():
        emb = jnp.maximum(
            jnp.dot(amg_ref[...], wm_ref[...],
                    preferred_element_type=jnp.float32)
            + jnp.dot(bg_ref[...], we_ref[...],
                      preferred_element_type=jnp.float32)
            + jnp.dot(hmr_ref[...], ws_ref[...],
                      preferred_element_type=jnp.float32), 0.0)
        hn = jnp.maximum(
            jnp.dot(emb, nw1_ref[...], preferred_element_type=jnp.float32)
            + nb1_ref[...], 0.0)
        pred = jnp.dot(hn, nw2_ref[...],
                       preferred_element_type=jnp.float32) + nb2_ref[...]
        row = lax.broadcasted_iota(jnp.int32, (NM_PAD, D), 0)
        col = lax.broadcasted_iota(jnp.int32, (NM_PAD, D), 1)
        err = jnp.where((row < NMASK) & (col < 2),
                        jnp.abs(pred - oinj_ref[...]), 0.0)
        acc[0] = jnp.sum(err)
        acc[1] = 0.0

    @pl.when(pid > 0)
    def _():
        he = jnp.maximum(r_ref[...] + eb1_ref[...], 0.0)
        pred = jnp.dot(he, ew2_ref[...],
                       preferred_element_type=jnp.float32) + eb2_ref[...]
        row = (pid - 1) * _EBLK + lax.broadcasted_iota(
            jnp.int32, (_EBLK, D), 0)
        col = lax.broadcasted_iota(jnp.int32, (_EBLK, D), 1)
        err = jnp.where((row < EMASK) & (col < 2),
                        jnp.abs(pred - oef_ref[...]), 0.0)
        acc[1] = acc[1] + jnp.sum(err)

    out_ref[0, 0] = 0.5 * acc[0] / (NMASK * 2) + 0.5 * acc[1] / (EMASK * 2)


def kernel(x, edge_index, edge_attr, W_in, b_in, W_msg, W_edge, W_self,
           nm_W1, nm_b1, nm_W2, nm_b2, node_mask_token,
           em_W1, em_b1, em_W2, em_b2, edge_mask_token):
    f32 = jnp.float32
    (idx_n_np, idx_e_np, eflag_np, nflag_np,
     rowids_np, off8_np) = _mask_indices()
    idx_n_pad = jnp.asarray(idx_n_np)

    src = edge_index[0]
    dst = edge_index[1]
    ea_bits = lax.bitcast_convert_type(edge_attr[:, :2], jnp.int32)
    epack32 = jnp.reshape(
        jnp.concatenate([src[:, None], dst[:, None], ea_bits], axis=1),
        (E // 32, D))
    ea16 = jnp.pad(edge_attr, ((0, 0), (0, 12)))
    tokrow = jnp.concatenate([edge_mask_token, jnp.zeros((14,), f32)])[None, :]
    colmask = (lax.broadcasted_iota(jnp.int32, (1, 16), 1) < 2)
    eam16 = jnp.where(jnp.asarray(eflag_np) & colmask, tokrow, ea16)
    x_pad = jnp.pad(x, ((0, NP - N), (0, 0)))
    ntokrow = jnp.concatenate(
        [node_mask_token, jnp.zeros((D - 2,), f32)])[None, :]
    ncolmask = (lax.broadcasted_iota(jnp.int32, (1, D), 1) < 2)
    xm = jnp.where(jnp.asarray(nflag_np) & ncolmask, ntokrow, x)
    xm_pad = jnp.pad(xm, ((0, NP - N), (0, 0)))

    xg, G = _k1()(x_pad, epack32, idx_n_pad, jnp.asarray(rowids_np))

    recs = pl.pallas_call(
        _kx_body, grid=(_NEB,),
        in_specs=[pl.BlockSpec((_EBLK, D), lambda i: (i, 0)),
                  pl.BlockSpec((_EBLK, 8), lambda i: (i, 0))],
        out_specs=pl.BlockSpec((_EBLK, D), lambda i: (i, 0)),
        out_shape=jax.ShapeDtypeStruct((EM_PAD, D), jnp.int32))(
            G, jnp.asarray(off8_np))
    srcm = recs[:, 0]
    dstm = recs[:, 1]

    b_row = b_in[None, :]
    h_pad = pl.pallas_call(
        _k2a_body, grid=(16,),
        in_specs=[pl.BlockSpec((632, D), lambda i: (i, 0)),
                  pl.BlockSpec((D, D), lambda i: (0, 0)),
                  pl.BlockSpec((1, D), lambda i: (0, 0))],
        out_specs=pl.BlockSpec((632, D), lambda i: (i, 0)),
        out_shape=jax.ShapeDtypeStruct((NP, D), f32))(x_pad, W_in, b_row)

    hm_pad = pl.pallas_call(
        _k2a_body, grid=(16,),
        in_specs=[pl.BlockSpec((632, D), lambda i: (i, 0)),
                  pl.BlockSpec((D, D), lambda i: (0, 0)),
                  pl.BlockSpec((1, D), lambda i: (0, 0))],
        out_specs=pl.BlockSpec((632, D), lambda i: (i, 0)),
        out_shape=jax.ShapeDtypeStruct((NP, D), f32))(xm_pad, W_in, b_row)

    Bm128, B128 = _k4b()(dst, jnp.reshape(ea16, (E * 16 // D, D)),
                         jnp.reshape(eam16, (E * 16 // D, D)))
    Bm = Bm128[:, :16]
    A, Amg = _k4()(h_pad, hm_pad, idx_n_pad, src, dst)

    W_edge16 = jnp.pad(W_edge, ((0, 12), (0, 0)))
    W1a = em_W1[:D]
    W1b = em_W1[D:]
    P, Q = pl.pallas_call(
        _k5_body, grid=(16,),
        in_specs=[pl.BlockSpec((632, D), lambda i: (i, 0)),
                  pl.BlockSpec((632, 16), lambda i: (i, 0)),
                  pl.BlockSpec((632, D), lambda i: (i, 0)),
                  pl.BlockSpec((D, D), lambda i: (0, 0)),
                  pl.BlockSpec((16, D), lambda i: (0, 0)),
                  pl.BlockSpec((D, D), lambda i: (0, 0)),
                  pl.BlockSpec((D, D), lambda i: (0, 0)),
                  pl.BlockSpec((D, D), lambda i: (0, 0))],
        out_specs=[pl.BlockSpec((632, D), lambda i: (i, 0)),
                   pl.BlockSpec((632, D), lambda i: (i, 0))],
        out_shape=(jax.ShapeDtypeStruct((NP, D), f32),
                   jax.ShapeDtypeStruct((NP, D), f32)))(
            A, Bm, h_pad, W_msg, W_edge16, W_self, W1a, W1b)

    R, hmg, Bg128 = _k6b()(srcm, dstm, P, Q, hm_pad, B128, idx_n_pad)
    Bg = Bg128[:, :16]

    orig_ef = lax.bitcast_convert_type(recs[:EMASK, 2:4], f32)
    oinj_pad = jnp.pad(xg[:, :2], ((0, 0), (0, D - 2)))
    nm_W2p = jnp.pad(nm_W2, ((0, 0), (0, D - 2)))
    nm_b2p = jnp.pad(nm_b2, (0, D - 2))[None, :]
    em_W2p = jnp.pad(em_W2, ((0, 0), (0, D - 2)))
    em_b2p = jnp.pad(em_b2, (0, D - 2))[None, :]
    oef_pad = jnp.pad(orig_ef, ((0, EM_PAD - EMASK), (0, D - 2)))

    const = lambda i: (0, 0)
    eblk = lambda i: (jnp.maximum(i - 1, 0), 0)
    out = pl.pallas_call(
        _k7_body, grid=(1 + _NEB,),
        in_specs=[pl.BlockSpec((NM_PAD, D), const),
                  pl.BlockSpec((NM_PAD, 16), const),
                  pl.BlockSpec((NM_PAD, D), const),
                  pl.BlockSpec((NM_PAD, D), const),
                  pl.BlockSpec((D, D), const),
                  pl.BlockSpec((16, D), const),
                  pl.BlockSpec((D, D), const),
                  pl.BlockSpec((D, D), const),
                  pl.BlockSpec((1, D), const),
                  pl.BlockSpec((D, D), const),
                  pl.BlockSpec((1, D), const),
                  pl.BlockSpec((_EBLK, D), eblk),
                  pl.BlockSpec((_EBLK, D), eblk),
                  pl.BlockSpec((1, D), const),
                  pl.BlockSpec((D, D), const),
                  pl.BlockSpec((1, D), const)],
        out_specs=pl.BlockSpec(memory_space=pltpu.SMEM),
        out_shape=jax.ShapeDtypeStruct((1, 1), f32),
        scratch_shapes=[pltpu.SMEM((2,), f32)])(
            Amg, Bg, hmg, oinj_pad, W_msg, W_edge16, W_self,
            nm_W1, nm_b1[None, :], nm_W2p, nm_b2p,
            R, oef_pad, em_b1[None, :], em_W2p, em_b2p)

    return out[0, 0]
